# baseline (speedup 1.0000x reference)
import functools

import jax
import jax.numpy as jnp
from jax import lax
from jax.experimental import pallas as pl
from jax.experimental.pallas import tpu as pltpu
from jax.experimental.pallas import tpu_sc as plsc

B, S, D = 4, 8192, 1024
NC, NS = 2, 16
NW = NC * NS
SEQ_PER_W = S // NW
C = 32
NCHUNK = SEQ_PER_W // C
CHUNK = C * D
U = 16
NVEC = CHUNK // 16

_mesh = plsc.VectorSubcoreMesh(core_axis_name="c", subcore_axis_name="s")


@functools.partial(
    pl.kernel,
    mesh=_mesh,
    out_type=jax.ShapeDtypeStruct((B * S * D,), jnp.float32),
    scratch_types=[
        pltpu.VMEM((CHUNK,), jnp.float32),
        pltpu.VMEM((CHUNK,), jnp.float32),
    ],
)
def _sc_add(x_hbm, pos_hbm, out_hbm, pos_v, x_v):
    wid = lax.axis_index("s") * NC + lax.axis_index("c")
    seq0 = wid * SEQ_PER_W

    def chunk_body(c, carry):
        off_pos = (seq0 + c * C) * D
        pltpu.sync_copy(pos_hbm.at[pl.ds(off_pos, CHUNK)], pos_v)

        def batch_body(b, carry):
            off_x = b * (S * D) + off_pos
            pltpu.sync_copy(x_hbm.at[pl.ds(off_x, CHUNK)], x_v)

            def add_body(k, carry):
                base = pl.multiple_of(k * (16 * U), 16 * U)
                for u in range(U):
                    sl = pl.ds(base + u * 16, 16)
                    plsc.addupdate(x_v.at[sl], pos_v[sl])
                return carry

            lax.fori_loop(0, NVEC // U, add_body, 0)
            pltpu.sync_copy(x_v, out_hbm.at[pl.ds(off_x, CHUNK)])
            return carry

        return lax.fori_loop(0, B, batch_body, carry)

    lax.fori_loop(0, NCHUNK, chunk_body, 0)


def kernel(x, pos_table):
    out = _sc_add(x.reshape(-1), pos_table.reshape(-1))
    return out.reshape(x.shape)


# speedup vs baseline: 1.6563x; 1.6563x over previous
import functools

import jax
import jax.numpy as jnp
from jax import lax
from jax.experimental import pallas as pl
from jax.experimental.pallas import tpu as pltpu
from jax.experimental.pallas import tpu_sc as plsc

B, S, D = 4, 8192, 1024
NC, NS = 2, 16
NW = NC * NS
SEQ_PER_W = S // NW
R = 16
NCH = SEQ_PER_W // R
VPR = D // 16

_mesh = plsc.VectorSubcoreMesh(core_axis_name="c", subcore_axis_name="s")


@functools.partial(
    pl.kernel,
    mesh=_mesh,
    out_type=jax.ShapeDtypeStruct((B, S, D), jnp.float32),
    scratch_types=[
        pltpu.VMEM((R, D), jnp.float32),
        pltpu.VMEM((R, D), jnp.float32),
        pltpu.VMEM((R, D), jnp.float32),
        pltpu.VMEM((R, D), jnp.float32),
        pltpu.VMEM((R, D), jnp.float32),
        pltpu.SemaphoreType.DMA,
        pltpu.SemaphoreType.DMA,
        pltpu.SemaphoreType.DMA,
        pltpu.SemaphoreType.DMA,
        pltpu.SemaphoreType.DMA,
        pltpu.SemaphoreType.DMA,
        pltpu.SemaphoreType.DMA,
        pltpu.SemaphoreType.DMA,
        pltpu.SemaphoreType.DMA,
    ],
)
def _sc_add(x_hbm, pos_hbm, out_hbm, pos_v, xb0, xb1, xb2, xb3,
            sp, si0, si1, si2, si3, so0, so1, so2, so3):
    bufs = (xb0, xb1, xb2, xb3)
    sins = (si0, si1, si2, si3)
    souts = (so0, so1, so2, so3)
    wid = lax.axis_index("s") * NC + lax.axis_index("c")
    seq0 = wid * SEQ_PER_W

    pltpu.async_copy(pos_hbm.at[pl.ds(seq0, R)], pos_v, sp)
    for u in range(B):
        pltpu.async_copy(x_hbm.at[u, pl.ds(seq0, R)], bufs[u], sins[u])

    def chunk_body(i, carry):
        row0 = seq0 + i * R
        for u in range(B):
            pltpu.make_async_copy(
                x_hbm.at[u, pl.ds(row0, R)], bufs[u], sins[u]).wait()
            if u == 0:
                pltpu.make_async_copy(
                    pos_hbm.at[pl.ds(row0, R)], pos_v, sp).wait()

            def add_row(r, c2, _buf=bufs[u]):
                for j in range(VPR):
                    sl = pl.ds(j * 16, 16)
                    plsc.addupdate(_buf.at[r, sl], pos_v[r, sl])
                return c2

            lax.fori_loop(0, R, add_row, 0)
            pltpu.async_copy(bufs[u], out_hbm.at[u, pl.ds(row0, R)], souts[u])

            pu = (u - 1) % B
            if u == 0:
                @pl.when(i > 0)
                def _():
                    pltpu.make_async_copy(
                        bufs[pu], out_hbm.at[pu, pl.ds(row0, R)],
                        souts[pu]).wait()
                    pltpu.async_copy(
                        x_hbm.at[pu, pl.ds(row0, R)], bufs[pu], sins[pu])
            else:
                pltpu.make_async_copy(
                    bufs[pu], out_hbm.at[pu, pl.ds(row0, R)], souts[pu]).wait()

                @pl.when(i + 1 < NCH)
                def _():
                    pltpu.async_copy(
                        x_hbm.at[pu, pl.ds(row0 + R, R)], bufs[pu], sins[pu])
        @pl.when(i + 1 < NCH)
        def _():
            pltpu.async_copy(pos_hbm.at[pl.ds(row0 + R, R)], pos_v, sp)
        return carry

    lax.fori_loop(0, NCH, chunk_body, 0)
    pltpu.make_async_copy(
        xb3, out_hbm.at[B - 1, pl.ds(seq0 + (NCH - 1) * R, R)], so3).wait()


def kernel(x, pos_table):
    return _sc_add(x, pos_table)


# speedup vs baseline: 3.3457x; 2.0200x over previous
import functools

import jax
import jax.numpy as jnp
from jax import lax
from jax.experimental import pallas as pl
from jax.experimental.pallas import tpu as pltpu
from jax.experimental.pallas import tpu_sc as plsc

B, S, D = 4, 8192, 1024
NC, NS = 2, 16
NW = NC * NS
SEQ_PER_W = S // NW
R = 16
NCH = SEQ_PER_W // R
VPR = D // 16

_mesh = plsc.VectorSubcoreMesh(core_axis_name="c", subcore_axis_name="s")


@functools.partial(
    pl.kernel,
    mesh=_mesh,
    out_type=jax.ShapeDtypeStruct((B, S, D), jnp.float32),
    scratch_types=[
        pltpu.VMEM((R, D), jnp.float32),
        pltpu.VMEM((R, D), jnp.float32),
        pltpu.VMEM((R, D), jnp.float32),
        pltpu.VMEM((R, D), jnp.float32),
        pltpu.VMEM((R, D), jnp.float32),
        pltpu.SemaphoreType.DMA,
        pltpu.SemaphoreType.DMA,
        pltpu.SemaphoreType.DMA,
        pltpu.SemaphoreType.DMA,
        pltpu.SemaphoreType.DMA,
        pltpu.SemaphoreType.DMA,
        pltpu.SemaphoreType.DMA,
        pltpu.SemaphoreType.DMA,
        pltpu.SemaphoreType.DMA,
    ],
)
def _sc_add(x_hbm, pos_hbm, out_hbm, pos_v, xb0, xb1, xb2, xb3,
            sp, si0, si1, si2, si3, so0, so1, so2, so3):
    bufs = (xb0, xb1, xb2, xb3)
    sins = (si0, si1, si2, si3)
    souts = (so0, so1, so2, so3)
    wid = lax.axis_index("s") * NC + lax.axis_index("c")
    seq0 = wid * SEQ_PER_W

    pltpu.async_copy(pos_hbm.at[pl.ds(seq0, R)], pos_v, sp)
    for u in range(B):
        pltpu.async_copy(x_hbm.at[u, pl.ds(seq0, R)], bufs[u], sins[u])

    def chunk_body(i, carry):
        row0 = seq0 + i * R
        for u in range(B):
            pltpu.make_async_copy(
                x_hbm.at[u, pl.ds(row0, R)], bufs[u], sins[u]).wait()
            if u == 0:
                pltpu.make_async_copy(
                    pos_hbm.at[pl.ds(row0, R)], pos_v, sp).wait()

            def add_row(r, c2, _buf=bufs[u]):
                G = 8
                NG = VPR // G

                def slices(g):
                    return [pl.ds((g * G + j) * 16, 16) for j in range(G)]

                cur_sls = slices(0)
                cur_vals = [pos_v[r, sl] for sl in cur_sls]
                for g in range(NG):
                    if g + 1 < NG:
                        nxt_sls = slices(g + 1)
                        nxt_vals = [pos_v[r, sl] for sl in nxt_sls]
                    for sl, v in zip(cur_sls, cur_vals):
                        plsc.addupdate(_buf.at[r, sl], v)
                    if g + 1 < NG:
                        cur_sls, cur_vals = nxt_sls, nxt_vals
                return c2

            lax.fori_loop(0, R, add_row, 0)
            pltpu.async_copy(bufs[u], out_hbm.at[u, pl.ds(row0, R)], souts[u])

            pu = (u - 1) % B
            if u == 0:
                @pl.when(i > 0)
                def _():
                    pltpu.make_async_copy(
                        bufs[pu], out_hbm.at[pu, pl.ds(row0, R)],
                        souts[pu]).wait()
                    pltpu.async_copy(
                        x_hbm.at[pu, pl.ds(row0, R)], bufs[pu], sins[pu])
            else:
                pltpu.make_async_copy(
                    bufs[pu], out_hbm.at[pu, pl.ds(row0, R)], souts[pu]).wait()

                @pl.when(i + 1 < NCH)
                def _():
                    pltpu.async_copy(
                        x_hbm.at[pu, pl.ds(row0 + R, R)], bufs[pu], sins[pu])
        @pl.when(i + 1 < NCH)
        def _():
            pltpu.async_copy(pos_hbm.at[pl.ds(row0 + R, R)], pos_v, sp)
        return carry

    lax.fori_loop(0, NCH, chunk_body, 0)
    pltpu.make_async_copy(
        xb3, out_hbm.at[B - 1, pl.ds(seq0 + (NCH - 1) * R, R)], so3).wait()


def kernel(x, pos_table):
    return _sc_add(x, pos_table)


# speedup vs baseline: 3.7003x; 1.1060x over previous
import functools

import jax
import jax.numpy as jnp
from jax import lax
from jax.experimental import pallas as pl
from jax.experimental.pallas import tpu as pltpu
from jax.experimental.pallas import tpu_sc as plsc

B, S, D = 4, 8192, 1024
NC, NS = 2, 16
NW = NC * NS
SEQ_PER_W = S // NW
R = 16
NCH = SEQ_PER_W // R
NPAIR = NCH // 2
VPR = D // 16

_mesh = plsc.VectorSubcoreMesh(core_axis_name="c", subcore_axis_name="s")


@functools.partial(
    pl.kernel,
    mesh=_mesh,
    out_type=jax.ShapeDtypeStruct((B, S, D), jnp.float32),
    scratch_types=[
        pltpu.VMEM((R, D), jnp.float32),
        pltpu.VMEM((R, D), jnp.float32),
        pltpu.VMEM((R, D), jnp.float32),
        pltpu.VMEM((R, D), jnp.float32),
        pltpu.VMEM((R, D), jnp.float32),
        pltpu.VMEM((R, D), jnp.float32),
        pltpu.SemaphoreType.DMA,
        pltpu.SemaphoreType.DMA,
        pltpu.SemaphoreType.DMA,
        pltpu.SemaphoreType.DMA,
        pltpu.SemaphoreType.DMA,
        pltpu.SemaphoreType.DMA,
        pltpu.SemaphoreType.DMA,
        pltpu.SemaphoreType.DMA,
        pltpu.SemaphoreType.DMA,
        pltpu.SemaphoreType.DMA,
    ],
)
def _sc_add(x_hbm, pos_hbm, out_hbm, pos_a, pos_b, xb0, xb1, xb2, xb3,
            spa, spb, si0, si1, si2, si3, so0, so1, so2, so3):
    bufs = (xb0, xb1, xb2, xb3)
    sins = (si0, si1, si2, si3)
    souts = (so0, so1, so2, so3)
    poss = (pos_a, pos_b)
    sps = (spa, spb)
    wid = lax.axis_index("s") * NC + lax.axis_index("c")
    seq0 = wid * SEQ_PER_W

    pltpu.async_copy(pos_hbm.at[pl.ds(seq0, R)], pos_a, spa)
    pltpu.async_copy(pos_hbm.at[pl.ds(seq0 + R, R)], pos_b, spb)
    for u in range(B):
        pltpu.async_copy(x_hbm.at[u, pl.ds(seq0, R)], bufs[u], sins[u])

    def pair_body(i2, carry):
        for half in range(2):
            row0 = seq0 + (2 * i2 + half) * R
            pos_v = poss[half]
            for u in range(B):
                pltpu.make_async_copy(
                    x_hbm.at[u, pl.ds(row0, R)], bufs[u], sins[u]).wait()
                if u == 0:
                    pltpu.make_async_copy(
                        pos_hbm.at[pl.ds(row0, R)], pos_v, sps[half]).wait()

                def add_row(r, c2, _buf=bufs[u], _pos=pos_v):
                    G = 16
                    NG = VPR // G

                    def slices(g):
                        return [pl.ds((g * G + j) * 16, 16) for j in range(G)]

                    cur_sls = slices(0)
                    cur_vals = [_pos[r, sl] for sl in cur_sls]
                    for g in range(NG):
                        if g + 1 < NG:
                            nxt_sls = slices(g + 1)
                            nxt_vals = [_pos[r, sl] for sl in nxt_sls]
                        for sl, v in zip(cur_sls, cur_vals):
                            plsc.addupdate(_buf.at[r, sl], v)
                        if g + 1 < NG:
                            cur_sls, cur_vals = nxt_sls, nxt_vals
                    return c2

                lax.fori_loop(0, R, add_row, 0)
                pltpu.async_copy(
                    bufs[u], out_hbm.at[u, pl.ds(row0, R)], souts[u])

                pu = (u - 1) % B
                if u == 0:
                    if half == 0:
                        @pl.when(i2 > 0)
                        def _():
                            pltpu.make_async_copy(
                                bufs[pu], out_hbm.at[pu, pl.ds(row0, R)],
                                souts[pu]).wait()
                            pltpu.async_copy(
                                x_hbm.at[pu, pl.ds(row0, R)],
                                bufs[pu], sins[pu])
                    else:
                        pltpu.make_async_copy(
                            bufs[pu], out_hbm.at[pu, pl.ds(row0, R)],
                            souts[pu]).wait()
                        pltpu.async_copy(
                            x_hbm.at[pu, pl.ds(row0, R)], bufs[pu], sins[pu])
                else:
                    pltpu.make_async_copy(
                        bufs[pu], out_hbm.at[pu, pl.ds(row0, R)],
                        souts[pu]).wait()
                    if half == 0:
                        pltpu.async_copy(
                            x_hbm.at[pu, pl.ds(row0 + R, R)],
                            bufs[pu], sins[pu])
                    else:
                        @pl.when(i2 + 1 < NPAIR)
                        def _():
                            pltpu.async_copy(
                                x_hbm.at[pu, pl.ds(row0 + R, R)],
                                bufs[pu], sins[pu])
            @pl.when(i2 + 1 < NPAIR)
            def _(_row0=row0, _half=half):
                pltpu.async_copy(
                    pos_hbm.at[pl.ds(_row0 + 2 * R, R)], poss[_half],
                    sps[_half])
        return carry

    lax.fori_loop(0, NPAIR, pair_body, 0)
    pltpu.make_async_copy(
        xb3, out_hbm.at[B - 1, pl.ds(seq0 + (NCH - 1) * R, R)], so3).wait()


def kernel(x, pos_table):
    return _sc_add(x, pos_table)
